# baseline (speedup 1.0000x reference)
import functools

import jax
import jax.numpy as jnp
from jax.experimental import pallas as pl
from jax.experimental.pallas import tpu as pltpu

B, S, D = 1, 2048, 768
E, K, DFF = 8, 2, 1024
T = B * S
TB = 128
NTB = T // TB


def _router_kernel(x_ref, wr_ref, w_ref):
    x = x_ref[...]
    lane = jax.lax.broadcasted_iota(jnp.int32, (TB, E), 1)
    l = jnp.dot(x, wr_ref[...], preferred_element_type=jnp.float32)
    l = jnp.where(lane < E - 1, l, -1e30)
    m1 = jnp.max(l, axis=1, keepdims=True)
    idx1 = jnp.min(jnp.where(l == m1, lane, E + 9), axis=1, keepdims=True)
    l2 = jnp.where(lane == idx1, -1e30, l)
    m2 = jnp.max(l2, axis=1, keepdims=True)
    idx2 = jnp.min(jnp.where(l2 == m2, lane, E + 9), axis=1, keepdims=True)
    p2 = jnp.exp(m2 - m1)
    denom = 1.0 + p2
    wfull = jnp.where(lane == idx1 + 1, 1.0 / denom, 0.0)
    wfull = wfull + jnp.where(lane == idx2 + 1, p2 / denom, 0.0)
    wfull = wfull + jnp.where(lane == 0, 1.0, 0.0)
    w_ref[...] = wfull


def _moe_dense_kernel(x_ref, w1_ref, w2_ref, w_ref, out_ref):
    e = pl.program_id(0)
    tb = pl.program_id(1)

    x = x_ref[...]
    h = jnp.dot(x, w1_ref[0], preferred_element_type=jnp.float32)
    h = h * jax.lax.logistic(h)
    y = jnp.dot(h.astype(jnp.bfloat16), w2_ref[0],
                preferred_element_type=jnp.float32)

    lane = jax.lax.broadcasted_iota(jnp.int32, (TB, E), 1)
    wcol = jnp.sum(jnp.where(lane == e, w_ref[...], 0.0), axis=1, keepdims=True)
    contrib = y * wcol

    @pl.when(e == 0)
    def _init():
        out_ref[pl.ds(tb * TB, TB), :] = contrib

    @pl.when(e > 0)
    def _acc():
        out_ref[pl.ds(tb * TB, TB), :] += contrib


def kernel(hidden_states, Wr, W1, W2, interpret=False):
    x = hidden_states.reshape(T, D)
    wr_pad = jnp.zeros((D, E), jnp.float32).at[:, : E - 1].set(Wr)
    x16 = x.astype(jnp.bfloat16)
    w1b = W1.astype(jnp.bfloat16)
    w2b = W2.astype(jnp.bfloat16)

    wfull = pl.pallas_call(
        _router_kernel,
        grid=(NTB,),
        in_specs=[
            pl.BlockSpec((TB, D), lambda tb: (tb, 0)),
            pl.BlockSpec((D, E), lambda tb: (0, 0)),
        ],
        out_specs=pl.BlockSpec((TB, E), lambda tb: (tb, 0)),
        out_shape=jax.ShapeDtypeStruct((T, E), jnp.float32),
        interpret=interpret,
    )(x, wr_pad)

    out = pl.pallas_call(
        _moe_dense_kernel,
        grid=(E, NTB),
        in_specs=[
            pl.BlockSpec((TB, D), lambda e, tb: (tb, 0)),
            pl.BlockSpec((1, D, DFF), lambda e, tb: (e, 0, 0)),
            pl.BlockSpec((1, DFF, D), lambda e, tb: (e, 0, 0)),
            pl.BlockSpec((TB, E), lambda e, tb: (tb, 0)),
        ],
        out_specs=pl.BlockSpec((T, D), lambda e, tb: (0, 0)),
        out_shape=jax.ShapeDtypeStruct((T, D), jnp.float32),
        interpret=interpret,
    )(x16, w1b, w2b, wfull)
    return out.reshape(B, S, D)


# speedup vs baseline: 1.2824x; 1.2824x over previous
import functools

import jax
import jax.numpy as jnp
from jax.experimental import pallas as pl
from jax.experimental.pallas import tpu as pltpu

B, S, D = 1, 2048, 768
E, K, DFF = 8, 2, 1024
T = B * S
TB = 256
NTB = T // TB


def _router_kernel(x_ref, wr_ref, w_ref):
    x = x_ref[...]
    lane = jax.lax.broadcasted_iota(jnp.int32, (TB, E), 1)
    l = jnp.dot(x, wr_ref[...], preferred_element_type=jnp.float32)
    l = jnp.where(lane < E - 1, l, -1e30)
    m1 = jnp.max(l, axis=1, keepdims=True)
    idx1 = jnp.min(jnp.where(l == m1, lane, E + 9), axis=1, keepdims=True)
    l2 = jnp.where(lane == idx1, -1e30, l)
    m2 = jnp.max(l2, axis=1, keepdims=True)
    idx2 = jnp.min(jnp.where(l2 == m2, lane, E + 9), axis=1, keepdims=True)
    p2 = jnp.exp(m2 - m1)
    denom = 1.0 + p2
    wfull = jnp.where(lane == idx1 + 1, 1.0 / denom, 0.0)
    wfull = wfull + jnp.where(lane == idx2 + 1, p2 / denom, 0.0)
    wfull = wfull + jnp.where(lane == 0, 1.0, 0.0)
    w_ref[...] = wfull


def _moe_dense_kernel(x_ref, w1_ref, w2_ref, w_ref, out_ref):
    e = pl.program_id(0)
    tb = pl.program_id(1)

    x = x_ref[...]
    h = jnp.dot(x, w1_ref[0], preferred_element_type=jnp.float32)
    h = h * jax.lax.logistic(h)
    y = jnp.dot(h.astype(jnp.bfloat16), w2_ref[0],
                preferred_element_type=jnp.float32)

    lane = jax.lax.broadcasted_iota(jnp.int32, (TB, E), 1)
    wcol = jnp.sum(jnp.where(lane == e, w_ref[...], 0.0), axis=1, keepdims=True)
    contrib = y * wcol

    @pl.when(e == 0)
    def _init():
        out_ref[pl.ds(tb * TB, TB), :] = contrib

    @pl.when(e > 0)
    def _acc():
        out_ref[pl.ds(tb * TB, TB), :] += contrib


def kernel(hidden_states, Wr, W1, W2, interpret=False):
    x = hidden_states.reshape(T, D)
    wr_pad = jnp.zeros((D, E), jnp.float32).at[:, : E - 1].set(Wr)
    x16 = x.astype(jnp.bfloat16)
    w1b = W1.astype(jnp.bfloat16)
    w2b = W2.astype(jnp.bfloat16)

    wfull = pl.pallas_call(
        _router_kernel,
        grid=(NTB,),
        in_specs=[
            pl.BlockSpec((TB, D), lambda tb: (tb, 0)),
            pl.BlockSpec((D, E), lambda tb: (0, 0)),
        ],
        out_specs=pl.BlockSpec((TB, E), lambda tb: (tb, 0)),
        out_shape=jax.ShapeDtypeStruct((T, E), jnp.float32),
        interpret=interpret,
    )(x, wr_pad)

    out = pl.pallas_call(
        _moe_dense_kernel,
        grid=(E, NTB),
        in_specs=[
            pl.BlockSpec((TB, D), lambda e, tb: (tb, 0)),
            pl.BlockSpec((1, D, DFF), lambda e, tb: (e, 0, 0)),
            pl.BlockSpec((1, DFF, D), lambda e, tb: (e, 0, 0)),
            pl.BlockSpec((TB, E), lambda e, tb: (tb, 0)),
        ],
        out_specs=pl.BlockSpec((T, D), lambda e, tb: (0, 0)),
        out_shape=jax.ShapeDtypeStruct((T, D), jnp.float32),
        interpret=interpret,
    )(x16, w1b, w2b, wfull)
    return out.reshape(B, S, D)


# speedup vs baseline: 1.5510x; 1.2095x over previous
import functools

import jax
import jax.numpy as jnp
from jax.experimental import pallas as pl
from jax.experimental.pallas import tpu as pltpu

B, S, D = 1, 2048, 768
E, K, DFF = 8, 2, 1024
T = B * S
TB = 512
NTB = T // TB


def _router_kernel(x_ref, wr_ref, w_ref):
    x = x_ref[...]
    lane = jax.lax.broadcasted_iota(jnp.int32, (TB, E), 1)
    l = jnp.dot(x, wr_ref[...], preferred_element_type=jnp.float32)
    l = jnp.where(lane < E - 1, l, -1e30)
    m1 = jnp.max(l, axis=1, keepdims=True)
    idx1 = jnp.min(jnp.where(l == m1, lane, E + 9), axis=1, keepdims=True)
    l2 = jnp.where(lane == idx1, -1e30, l)
    m2 = jnp.max(l2, axis=1, keepdims=True)
    idx2 = jnp.min(jnp.where(l2 == m2, lane, E + 9), axis=1, keepdims=True)
    p2 = jnp.exp(m2 - m1)
    denom = 1.0 + p2
    wfull = jnp.where(lane == idx1 + 1, 1.0 / denom, 0.0)
    wfull = wfull + jnp.where(lane == idx2 + 1, p2 / denom, 0.0)
    wfull = wfull + jnp.where(lane == 0, 1.0, 0.0)
    w_ref[...] = wfull


def _moe_dense_kernel(x_ref, w1_ref, w2_ref, w_ref, out_ref):
    e = pl.program_id(0)
    tb = pl.program_id(1)

    x = x_ref[...]
    h = jnp.dot(x, w1_ref[0], preferred_element_type=jnp.float32)
    h = h * jax.lax.logistic(h)
    y = jnp.dot(h.astype(jnp.bfloat16), w2_ref[0],
                preferred_element_type=jnp.float32)

    lane = jax.lax.broadcasted_iota(jnp.int32, (TB, E), 1)
    wcol = jnp.sum(jnp.where(lane == e, w_ref[...], 0.0), axis=1, keepdims=True)
    contrib = y * wcol

    @pl.when(e == 0)
    def _init():
        out_ref[pl.ds(tb * TB, TB), :] = contrib

    @pl.when(e > 0)
    def _acc():
        out_ref[pl.ds(tb * TB, TB), :] += contrib


def kernel(hidden_states, Wr, W1, W2, interpret=False):
    x = hidden_states.reshape(T, D)
    wr_pad = jnp.zeros((D, E), jnp.float32).at[:, : E - 1].set(Wr)
    x16 = x.astype(jnp.bfloat16)
    w1b = W1.astype(jnp.bfloat16)
    w2b = W2.astype(jnp.bfloat16)

    wfull = pl.pallas_call(
        _router_kernel,
        grid=(NTB,),
        in_specs=[
            pl.BlockSpec((TB, D), lambda tb: (tb, 0)),
            pl.BlockSpec((D, E), lambda tb: (0, 0)),
        ],
        out_specs=pl.BlockSpec((TB, E), lambda tb: (tb, 0)),
        out_shape=jax.ShapeDtypeStruct((T, E), jnp.float32),
        interpret=interpret,
    )(x, wr_pad)

    out = pl.pallas_call(
        _moe_dense_kernel,
        grid=(E, NTB),
        in_specs=[
            pl.BlockSpec((TB, D), lambda e, tb: (tb, 0)),
            pl.BlockSpec((1, D, DFF), lambda e, tb: (e, 0, 0)),
            pl.BlockSpec((1, DFF, D), lambda e, tb: (e, 0, 0)),
            pl.BlockSpec((TB, E), lambda e, tb: (tb, 0)),
        ],
        out_specs=pl.BlockSpec((T, D), lambda e, tb: (0, 0)),
        out_shape=jax.ShapeDtypeStruct((T, D), jnp.float32),
        interpret=interpret,
    )(x16, w1b, w2b, wfull)
    return out.reshape(B, S, D)


# speedup vs baseline: 1.6598x; 1.0701x over previous
import functools

import jax
import jax.numpy as jnp
from jax.experimental import pallas as pl
from jax.experimental.pallas import tpu as pltpu

B, S, D = 1, 2048, 768
E, K, DFF = 8, 2, 1024
T = B * S
TB = 1024
NTB = T // TB


def _router_kernel(x_ref, wr_ref, w_ref):
    x = x_ref[...]
    lane = jax.lax.broadcasted_iota(jnp.int32, (TB, E), 1)
    l = jnp.dot(x, wr_ref[...], preferred_element_type=jnp.float32)
    l = jnp.where(lane < E - 1, l, -1e30)
    m1 = jnp.max(l, axis=1, keepdims=True)
    idx1 = jnp.min(jnp.where(l == m1, lane, E + 9), axis=1, keepdims=True)
    l2 = jnp.where(lane == idx1, -1e30, l)
    m2 = jnp.max(l2, axis=1, keepdims=True)
    idx2 = jnp.min(jnp.where(l2 == m2, lane, E + 9), axis=1, keepdims=True)
    p2 = jnp.exp(m2 - m1)
    denom = 1.0 + p2
    wfull = jnp.where(lane == idx1 + 1, 1.0 / denom, 0.0)
    wfull = wfull + jnp.where(lane == idx2 + 1, p2 / denom, 0.0)
    wfull = wfull + jnp.where(lane == 0, 1.0, 0.0)
    w_ref[...] = wfull


def _moe_dense_kernel(x_ref, w1_ref, w2_ref, w_ref, out_ref):
    e = pl.program_id(0)
    tb = pl.program_id(1)

    x = x_ref[...]
    h = jnp.dot(x, w1_ref[0], preferred_element_type=jnp.float32)
    h = h * jax.lax.logistic(h)
    y = jnp.dot(h.astype(jnp.bfloat16), w2_ref[0],
                preferred_element_type=jnp.float32)

    lane = jax.lax.broadcasted_iota(jnp.int32, (TB, E), 1)
    wcol = jnp.sum(jnp.where(lane == e, w_ref[...], 0.0), axis=1, keepdims=True)
    contrib = y * wcol

    @pl.when(e == 0)
    def _init():
        out_ref[pl.ds(tb * TB, TB), :] = contrib

    @pl.when(e > 0)
    def _acc():
        out_ref[pl.ds(tb * TB, TB), :] += contrib


def kernel(hidden_states, Wr, W1, W2, interpret=False):
    x = hidden_states.reshape(T, D)
    wr_pad = jnp.zeros((D, E), jnp.float32).at[:, : E - 1].set(Wr)
    x16 = x.astype(jnp.bfloat16)
    w1b = W1.astype(jnp.bfloat16)
    w2b = W2.astype(jnp.bfloat16)

    wfull = pl.pallas_call(
        _router_kernel,
        grid=(NTB,),
        in_specs=[
            pl.BlockSpec((TB, D), lambda tb: (tb, 0)),
            pl.BlockSpec((D, E), lambda tb: (0, 0)),
        ],
        out_specs=pl.BlockSpec((TB, E), lambda tb: (tb, 0)),
        out_shape=jax.ShapeDtypeStruct((T, E), jnp.float32),
        interpret=interpret,
    )(x, wr_pad)

    out = pl.pallas_call(
        _moe_dense_kernel,
        grid=(E, NTB),
        in_specs=[
            pl.BlockSpec((TB, D), lambda e, tb: (tb, 0)),
            pl.BlockSpec((1, D, DFF), lambda e, tb: (e, 0, 0)),
            pl.BlockSpec((1, DFF, D), lambda e, tb: (e, 0, 0)),
            pl.BlockSpec((TB, E), lambda e, tb: (tb, 0)),
        ],
        out_specs=pl.BlockSpec((T, D), lambda e, tb: (0, 0)),
        out_shape=jax.ShapeDtypeStruct((T, D), jnp.float32),
        interpret=interpret,
    )(x16, w1b, w2b, wfull)
    return out.reshape(B, S, D)


# speedup vs baseline: 1.6604x; 1.0004x over previous
import functools

import jax
import jax.numpy as jnp
from jax.experimental import pallas as pl
from jax.experimental.pallas import tpu as pltpu

B, S, D = 1, 2048, 768
E, K, DFF = 8, 2, 1024
T = B * S
TB = 2048
NTB = T // TB


def _router_kernel(x_ref, wr_ref, w_ref):
    x = x_ref[...]
    lane = jax.lax.broadcasted_iota(jnp.int32, (TB, E), 1)
    l = jnp.dot(x, wr_ref[...], preferred_element_type=jnp.float32)
    l = jnp.where(lane < E - 1, l, -1e30)
    m1 = jnp.max(l, axis=1, keepdims=True)
    idx1 = jnp.min(jnp.where(l == m1, lane, E + 9), axis=1, keepdims=True)
    l2 = jnp.where(lane == idx1, -1e30, l)
    m2 = jnp.max(l2, axis=1, keepdims=True)
    idx2 = jnp.min(jnp.where(l2 == m2, lane, E + 9), axis=1, keepdims=True)
    p2 = jnp.exp(m2 - m1)
    denom = 1.0 + p2
    wfull = jnp.where(lane == idx1 + 1, 1.0 / denom, 0.0)
    wfull = wfull + jnp.where(lane == idx2 + 1, p2 / denom, 0.0)
    wfull = wfull + jnp.where(lane == 0, 1.0, 0.0)
    w_ref[...] = wfull


def _moe_dense_kernel(x_ref, w1_ref, w2_ref, w_ref, out_ref):
    e = pl.program_id(0)
    tb = pl.program_id(1)

    x = x_ref[...]
    h = jnp.dot(x, w1_ref[0], preferred_element_type=jnp.float32)
    h = h * jax.lax.logistic(h)
    y = jnp.dot(h.astype(jnp.bfloat16), w2_ref[0],
                preferred_element_type=jnp.float32)

    lane = jax.lax.broadcasted_iota(jnp.int32, (TB, E), 1)
    wcol = jnp.sum(jnp.where(lane == e, w_ref[...], 0.0), axis=1, keepdims=True)
    contrib = y * wcol

    @pl.when(e == 0)
    def _init():
        out_ref[pl.ds(tb * TB, TB), :] = contrib

    @pl.when(e > 0)
    def _acc():
        out_ref[pl.ds(tb * TB, TB), :] += contrib


def kernel(hidden_states, Wr, W1, W2, interpret=False):
    x = hidden_states.reshape(T, D)
    wr_pad = jnp.zeros((D, E), jnp.float32).at[:, : E - 1].set(Wr)
    x16 = x.astype(jnp.bfloat16)
    w1b = W1.astype(jnp.bfloat16)
    w2b = W2.astype(jnp.bfloat16)

    wfull = pl.pallas_call(
        _router_kernel,
        grid=(NTB,),
        in_specs=[
            pl.BlockSpec((TB, D), lambda tb: (tb, 0)),
            pl.BlockSpec((D, E), lambda tb: (0, 0)),
        ],
        out_specs=pl.BlockSpec((TB, E), lambda tb: (tb, 0)),
        out_shape=jax.ShapeDtypeStruct((T, E), jnp.float32),
        interpret=interpret,
    )(x, wr_pad)

    out = pl.pallas_call(
        _moe_dense_kernel,
        grid=(E, NTB),
        in_specs=[
            pl.BlockSpec((TB, D), lambda e, tb: (tb, 0)),
            pl.BlockSpec((1, D, DFF), lambda e, tb: (e, 0, 0)),
            pl.BlockSpec((1, DFF, D), lambda e, tb: (e, 0, 0)),
            pl.BlockSpec((TB, E), lambda e, tb: (tb, 0)),
        ],
        out_specs=pl.BlockSpec((T, D), lambda e, tb: (0, 0)),
        out_shape=jax.ShapeDtypeStruct((T, D), jnp.float32),
        interpret=interpret,
    )(x16, w1b, w2b, wfull)
    return out.reshape(B, S, D)
